# Initial kernel scaffold; baseline (speedup 1.0000x reference)
import jax
import jax.numpy as jnp
from jax.experimental import pallas as pl


def kernel(coordinate, size, x_embedding, y_embedding):
    raise NotImplementedError("write your pallas kernel here")


# speedup vs baseline: 4.6248x; 4.6248x over previous
import functools

import jax
import jax.numpy as jnp
from jax import lax
from jax.experimental import pallas as pl
from jax.experimental.pallas import tpu as pltpu
from jax.experimental.pallas import tpu_sc as plsc

_RES_X = 1024
_RES_Y = 1024
_DH = 256
_B = 16 * 32 * 128
_NC = 2
_NS = 16
_L = 16
_NW = _NC * _NS
_BPW = _B // _NW
_CHUNK = 128
_NCHUNK = _BPW // _CHUNK
_ZROW = _RES_X


def _pos_enc_body(cx_hbm, cy_hbm, sz_hbm, xt_hbm, yt_hbm, out_hbm,
                  s_v, cx_v, cy_v, ix_v, iy_v, xrows_v, yrows_v,
                  semx, semy):
    wid = lax.axis_index("s") * _NC + lax.axis_index("c")
    base = wid * _BPW
    pltpu.sync_copy(sz_hbm, s_v)
    s_h = s_v[0, :]
    s_w = s_v[1, :]

    def chunk_body(k, carry):
        off = pl.multiple_of(base + k * _CHUNK, _CHUNK)
        pltpu.sync_copy(cx_hbm.at[pl.ds(off, _CHUNK)], cx_v)
        pltpu.sync_copy(cy_hbm.at[pl.ds(off, _CHUNK)], cy_v)
        for i in range(_CHUNK // _L):
            sl = pl.ds(i * _L, _L)
            x = cx_v[sl]
            y = cy_v[sl]
            ix = jnp.clip(((_RES_X * x) / s_w).astype(jnp.int32), 0, _RES_X - 1)
            iy = jnp.clip(((_RES_Y * y) / s_h).astype(jnp.int32), 0, _RES_Y - 1)
            neg = x < 0.0
            ix_v[sl] = jnp.where(neg, _ZROW, ix)
            iy_v[sl] = jnp.where(neg, _ZROW, iy)
        cpx = pltpu.async_copy(xt_hbm.at[ix_v], xrows_v, semx)
        cpy = pltpu.async_copy(yt_hbm.at[iy_v], yrows_v, semy)
        cpx.wait()
        pltpu.sync_copy(xrows_v, out_hbm.at[pl.ds(off, _CHUNK), pl.ds(0, _DH)])
        cpy.wait()
        pltpu.sync_copy(yrows_v, out_hbm.at[pl.ds(off, _CHUNK), pl.ds(_DH, _DH)])
        return carry

    lax.fori_loop(0, _NCHUNK, chunk_body, 0)


_pos_enc = functools.partial(
    pl.kernel,
    out_type=jax.ShapeDtypeStruct((_B, 2 * _DH), jnp.float32),
    mesh=plsc.VectorSubcoreMesh(core_axis_name="c", subcore_axis_name="s"),
    scratch_types=[
        pltpu.VMEM((2, _L), jnp.float32),
        pltpu.VMEM((_CHUNK,), jnp.float32),
        pltpu.VMEM((_CHUNK,), jnp.float32),
        pltpu.VMEM((_CHUNK,), jnp.int32),
        pltpu.VMEM((_CHUNK,), jnp.int32),
        pltpu.VMEM((_CHUNK, _DH), jnp.float32),
        pltpu.VMEM((_CHUNK, _DH), jnp.float32),
        pltpu.SemaphoreType.DMA,
        pltpu.SemaphoreType.DMA,
    ],
)(_pos_enc_body)


def kernel(coordinate, size, x_embedding, y_embedding):
    lead = coordinate.shape[:-1]
    cx = coordinate[..., 0].reshape(_B)
    cy = coordinate[..., 1].reshape(_B)
    zrow = jnp.zeros((8, _DH), x_embedding.dtype)
    xt = jnp.concatenate([x_embedding, zrow], axis=0)
    yt = jnp.concatenate([y_embedding, zrow], axis=0)
    svec = jnp.broadcast_to(size.astype(jnp.float32).reshape(2, 1), (2, _L))
    out = _pos_enc(cx, cy, svec, xt, yt)
    return out.reshape(*lead, 2 * _DH)


# speedup vs baseline: 5.0588x; 1.0938x over previous
import functools

import jax
import jax.numpy as jnp
from jax import lax
from jax.experimental import pallas as pl
from jax.experimental.pallas import tpu as pltpu
from jax.experimental.pallas import tpu_sc as plsc

_RES_X = 1024
_RES_Y = 1024
_DH = 256
_B = 16 * 32 * 128
_NC = 2
_NS = 16
_L = 16
_NW = _NC * _NS
_BPW = _B // _NW
_CHUNK = 64
_NCHUNK = _BPW // _CHUNK
_NJ = _NCHUNK // 2
_ZROW = _RES_X


def _pos_enc_body(cx_hbm, cy_hbm, sz_hbm, xt_hbm, yt_hbm, out_hbm,
                  s_v,
                  cx0, cy0, ix0, iy0, xr0, yr0,
                  cx1, cy1, ix1, iy1, xr1, yr1,
                  sx0, sy0, sx1, sy1):
    wid = lax.axis_index("s") * _NC + lax.axis_index("c")
    base = wid * _BPW
    pltpu.sync_copy(sz_hbm, s_v)
    s_h = s_v[0, :]
    s_w = s_v[1, :]

    slot0 = (cx0, cy0, ix0, iy0, xr0, yr0, sx0, sy0)
    slot1 = (cx1, cy1, ix1, iy1, xr1, yr1, sx1, sy1)

    def fire(slot, k):
        cx_v, cy_v, ix_v, iy_v, xr_v, yr_v, semx, semy = slot
        off = pl.multiple_of(base + k * _CHUNK, _CHUNK)
        pltpu.sync_copy(cx_hbm.at[pl.ds(off, _CHUNK)], cx_v)
        pltpu.sync_copy(cy_hbm.at[pl.ds(off, _CHUNK)], cy_v)
        for i in range(_CHUNK // _L):
            sl = pl.ds(i * _L, _L)
            x = cx_v[sl]
            y = cy_v[sl]
            ix = jnp.clip(((_RES_X * x) / s_w).astype(jnp.int32), 0, _RES_X - 1)
            iy = jnp.clip(((_RES_Y * y) / s_h).astype(jnp.int32), 0, _RES_Y - 1)
            neg = x < 0.0
            ix_v[sl] = jnp.where(neg, _ZROW, ix)
            iy_v[sl] = jnp.where(neg, _ZROW, iy)
        pltpu.async_copy(xt_hbm.at[ix_v], xr_v, semx)
        pltpu.async_copy(yt_hbm.at[iy_v], yr_v, semy)

    def drain(slot, k):
        cx_v, cy_v, ix_v, iy_v, xr_v, yr_v, semx, semy = slot
        off = pl.multiple_of(base + k * _CHUNK, _CHUNK)
        pltpu.make_async_copy(xt_hbm.at[ix_v], xr_v, semx).wait()
        pltpu.sync_copy(xr_v, out_hbm.at[pl.ds(off, _CHUNK), pl.ds(0, _DH)])
        pltpu.make_async_copy(yt_hbm.at[iy_v], yr_v, semy).wait()
        pltpu.sync_copy(yr_v, out_hbm.at[pl.ds(off, _CHUNK), pl.ds(_DH, _DH)])

    fire(slot0, 0)

    def body(j, carry):
        k0 = 2 * j
        fire(slot1, k0 + 1)
        drain(slot0, k0)

        @pl.when(j < _NJ - 1)
        def _():
            fire(slot0, k0 + 2)

        drain(slot1, k0 + 1)
        return carry

    lax.fori_loop(0, _NJ, body, 0)


_pos_enc = functools.partial(
    pl.kernel,
    out_type=jax.ShapeDtypeStruct((_B, 2 * _DH), jnp.float32),
    mesh=plsc.VectorSubcoreMesh(core_axis_name="c", subcore_axis_name="s"),
    scratch_types=[
        pltpu.VMEM((2, _L), jnp.float32),
        pltpu.VMEM((_CHUNK,), jnp.float32),
        pltpu.VMEM((_CHUNK,), jnp.float32),
        pltpu.VMEM((_CHUNK,), jnp.int32),
        pltpu.VMEM((_CHUNK,), jnp.int32),
        pltpu.VMEM((_CHUNK, _DH), jnp.float32),
        pltpu.VMEM((_CHUNK, _DH), jnp.float32),
        pltpu.VMEM((_CHUNK,), jnp.float32),
        pltpu.VMEM((_CHUNK,), jnp.float32),
        pltpu.VMEM((_CHUNK,), jnp.int32),
        pltpu.VMEM((_CHUNK,), jnp.int32),
        pltpu.VMEM((_CHUNK, _DH), jnp.float32),
        pltpu.VMEM((_CHUNK, _DH), jnp.float32),
        pltpu.SemaphoreType.DMA,
        pltpu.SemaphoreType.DMA,
        pltpu.SemaphoreType.DMA,
        pltpu.SemaphoreType.DMA,
    ],
)(_pos_enc_body)


def kernel(coordinate, size, x_embedding, y_embedding):
    lead = coordinate.shape[:-1]
    cx = coordinate[..., 0].reshape(_B)
    cy = coordinate[..., 1].reshape(_B)
    zrow = jnp.zeros((8, _DH), x_embedding.dtype)
    xt = jnp.concatenate([x_embedding, zrow], axis=0)
    yt = jnp.concatenate([y_embedding, zrow], axis=0)
    svec = jnp.broadcast_to(size.astype(jnp.float32).reshape(2, 1), (2, _L))
    out = _pos_enc(cx, cy, svec, xt, yt)
    return out.reshape(*lead, 2 * _DH)
